# baseline (speedup 1.0000x reference)
import functools

import jax
import jax.numpy as jnp
from jax import lax
from jax.experimental import pallas as pl
from jax.experimental.pallas import tpu as pltpu
from jax.experimental.pallas import tpu_sc as plsc

_NUM_CORES = 2
_NUM_SUBCORES = 16
_NUM_WORKERS = _NUM_CORES * _NUM_SUBCORES
_P1_CHUNK = 57216


@functools.partial(jax.jit, static_argnums=(2,))
def _gather_t(table_t, idx_t, v):
    d, _ = table_t.shape
    l_fields, b = idx_t.shape
    m_rows = l_fields * d
    mesh = plsc.VectorSubcoreMesh(core_axis_name="c", subcore_axis_name="s")

    chunks = [(t * _P1_CHUNK, _P1_CHUNK) for t in range(v // _P1_CHUNK)]
    rem_off = (v // _P1_CHUNK) * _P1_CHUNK
    rem_aligned = ((v - rem_off) // 128) * 128
    if rem_aligned:
        chunks.append((rem_off, rem_aligned))
    tail_off = rem_off + rem_aligned
    tail = v - tail_off

    @functools.partial(
        pl.kernel,
        mesh=mesh,
        out_type=[
            jax.ShapeDtypeStruct((m_rows, b), jnp.float32),
            jax.ShapeDtypeStruct((d * v,), jnp.float32),
        ],
        scratch_types=[
            pltpu.VMEM((b,), jnp.int32),
            pltpu.VMEM((b,), jnp.int32),
            pltpu.VMEM((b,), jnp.float32),
            pltpu.VMEM((b,), jnp.float32),
            pltpu.VMEM((_P1_CHUNK,), jnp.float32),
            pltpu.VMEM((_P1_CHUNK,), jnp.float32),
            pltpu.VMEM((1, max(tail, 1)), jnp.float32),
            pltpu.SemaphoreType.DMA,
            pltpu.SemaphoreType.DMA,
            pltpu.SemaphoreType.DMA,
            pltpu.SemaphoreType.DMA,
            pltpu.SemaphoreType.DMA,
            pltpu.SemaphoreType.DMA,
            pltpu.SemaphoreType.DMA,
        ],
        compiler_params=pltpu.CompilerParams(needs_layout_passes=False),
    )
    def gather(table_hbm, idx_hbm, out_hbm, scr_hbm, idx0_v, idx1_v, vals0_v,
               vals1_v, chunk0_v, chunk1_v, tail_v, s_in0, s_in1, s_out,
               s_idx, s_g, s_w0, s_w1):
        s_in = [s_in0, s_in1]
        s_w = [s_w0, s_w1]
        idx_b = [idx0_v, idx1_v]
        vals_b = [vals0_v, vals1_v]
        chunk_b = [chunk0_v, chunk1_v]
        w = lax.axis_index("s") * _NUM_CORES + lax.axis_index("c")
        row = scr_hbm.at[pl.ds(w * v, v)]
        feat = table_hbm.at[w]

        ci = [None] * l_fields
        for r in range(min(2, l_fields)):
            ci[r] = pltpu.async_copy(idx_hbm.at[r], idx_b[r % 2], s_idx)

        n_ch = len(chunks)
        for t in range(n_ch):
            off, sz = chunks[t]
            pltpu.sync_copy(
                feat.at[pl.ds(off, sz)], chunk_b[t % 2].at[pl.ds(0, sz)]
            )
            pltpu.sync_copy(
                chunk_b[t % 2].at[pl.ds(0, sz)], row.at[pl.ds(off, sz)]
            )
        if tail:
            pltpu.sync_copy(
                table_hbm.at[pl.ds(w, 1), pl.ds(tail_off, tail)], tail_v
            )
            pltpu.sync_copy(tail_v.at[0], row.at[pl.ds(tail_off, tail)])

        for r in range(l_fields):
            ci[r].wait()
            pltpu.async_copy(
                row.at[idx_b[r % 2]], vals_b[r % 2], s_g
            ).wait()
            if 1 <= r + 1 < l_fields and r >= 1:
                ci[r + 1] = pltpu.async_copy(
                    idx_hbm.at[r + 1], idx_b[(r + 1) % 2], s_idx
                )
            pltpu.sync_copy(vals_b[r % 2], out_hbm.at[r * d + w])

    return gather(table_t, idx_t)[0]


def kernel(inputs, embedding):
    b, l = inputs.shape
    v, d = embedding.shape
    idx_t = inputs.T.astype(jnp.int32)
    table_t = embedding.T
    out_t = _gather_t(table_t, idx_t, v)
    return out_t.T


# speedup vs baseline: 1.0509x; 1.0509x over previous
import functools

import jax
import jax.numpy as jnp
from jax import lax
from jax.experimental import pallas as pl
from jax.experimental.pallas import tpu as pltpu
from jax.experimental.pallas import tpu_sc as plsc

_NUM_CORES = 2
_NUM_SUBCORES = 16
_NUM_WORKERS = _NUM_CORES * _NUM_SUBCORES
_P1_CHUNK = 98304


@functools.partial(jax.jit, static_argnums=(2,))
def _gather_t(table_t, idx_t, v):
    d, _ = table_t.shape
    l_fields, b = idx_t.shape
    m_rows = l_fields * d
    mesh = plsc.VectorSubcoreMesh(core_axis_name="c", subcore_axis_name="s")

    chunks = [(t * _P1_CHUNK, _P1_CHUNK) for t in range(v // _P1_CHUNK)]
    rem_off = (v // _P1_CHUNK) * _P1_CHUNK
    rem_aligned = ((v - rem_off) // 128) * 128
    if rem_aligned:
        chunks.append((rem_off, rem_aligned))
    tail_off = rem_off + rem_aligned
    tail = v - tail_off

    @functools.partial(
        pl.kernel,
        mesh=mesh,
        out_type=[
            jax.ShapeDtypeStruct((m_rows, b), jnp.float32),
            jax.ShapeDtypeStruct((d * v,), jnp.float32),
        ],
        scratch_types=[
            pltpu.VMEM((b,), jnp.int32),
            pltpu.VMEM((b,), jnp.int32),
            pltpu.VMEM((b,), jnp.float32),
            pltpu.VMEM((b,), jnp.float32),
            pltpu.VMEM((_P1_CHUNK,), jnp.float32),
            pltpu.VMEM((1, max(tail, 1)), jnp.float32),
            pltpu.SemaphoreType.DMA,
            pltpu.SemaphoreType.DMA,
            pltpu.SemaphoreType.DMA,
            pltpu.SemaphoreType.DMA,
        ],
        compiler_params=pltpu.CompilerParams(needs_layout_passes=False),
    )
    def gather(table_hbm, idx_hbm, out_hbm, scr_hbm, idx0_v, idx1_v, vals0_v,
               vals1_v, chunk_v, tail_v, s_idx, s_g, s_w0, s_w1):
        idx_b = [idx0_v, idx1_v]
        vals_b = [vals0_v, vals1_v]
        s_w = [s_w0, s_w1]
        w = lax.axis_index("s") * _NUM_CORES + lax.axis_index("c")
        row = scr_hbm.at[pl.ds(w * v, v)]
        feat = table_hbm.at[w]

        ci = [None] * l_fields
        for r in range(min(2, l_fields)):
            ci[r] = pltpu.async_copy(idx_hbm.at[r], idx_b[r % 2], s_idx)

        for off, sz in chunks:
            pltpu.sync_copy(
                feat.at[pl.ds(off, sz)], chunk_v.at[pl.ds(0, sz)]
            )
            pltpu.sync_copy(
                chunk_v.at[pl.ds(0, sz)], row.at[pl.ds(off, sz)]
            )
        if tail:
            pltpu.sync_copy(
                table_hbm.at[pl.ds(w, 1), pl.ds(tail_off, tail)], tail_v
            )
            pltpu.sync_copy(tail_v.at[0], row.at[pl.ds(tail_off, tail)])

        cg = [None] * l_fields
        cw = [None] * l_fields
        for r in range(l_fields):
            ci[r].wait()
            if r >= 1:
                cg[r - 1].wait()
                cw[r - 1] = pltpu.async_copy(
                    vals_b[(r - 1) % 2], out_hbm.at[(r - 1) * d + w],
                    s_w[(r - 1) % 2]
                )
                if r + 1 < l_fields:
                    ci[r + 1] = pltpu.async_copy(
                        idx_hbm.at[r + 1], idx_b[(r + 1) % 2], s_idx
                    )
            if r >= 2:
                cw[r - 2].wait()
            cg[r] = pltpu.async_copy(
                row.at[idx_b[r % 2]], vals_b[r % 2], s_g
            )
        cg[l_fields - 1].wait()
        if l_fields >= 2:
            cw[l_fields - 2].wait()
        pltpu.sync_copy(
            vals_b[(l_fields - 1) % 2],
            out_hbm.at[(l_fields - 1) * d + w],
        )

    return gather(table_t, idx_t)[0]


def kernel(inputs, embedding):
    b, l = inputs.shape
    v, d = embedding.shape
    idx_t = inputs.T.astype(jnp.int32)
    table_t = embedding.T
    out_t = _gather_t(table_t, idx_t, v)
    return out_t.T


# speedup vs baseline: 1.1813x; 1.1241x over previous
import functools

import jax
import jax.numpy as jnp
from jax import lax
from jax.experimental import pallas as pl
from jax.experimental.pallas import tpu as pltpu
from jax.experimental.pallas import tpu_sc as plsc

_NUM_CORES = 2
_NUM_SUBCORES = 16
_NUM_WORKERS = _NUM_CORES * _NUM_SUBCORES
_P1_CHUNK = 98304


@functools.partial(jax.jit, static_argnums=(2,))
def _gather_t(table_t, idx_t, v):
    d, _ = table_t.shape
    l_fields, b = idx_t.shape
    m_rows = l_fields * d
    mesh = plsc.VectorSubcoreMesh(core_axis_name="c", subcore_axis_name="s")

    chunks = [(t * _P1_CHUNK, _P1_CHUNK) for t in range(v // _P1_CHUNK)]
    rem_off = (v // _P1_CHUNK) * _P1_CHUNK
    rem_aligned = ((v - rem_off) // 128) * 128
    if rem_aligned:
        chunks.append((rem_off, rem_aligned))
    tail_off = rem_off + rem_aligned
    tail = v - tail_off

    @functools.partial(
        pl.kernel,
        mesh=mesh,
        out_type=[
            jax.ShapeDtypeStruct((m_rows, b), jnp.float32),
            jax.ShapeDtypeStruct((d * v,), jnp.float32),
        ],
        scratch_types=[
            pltpu.VMEM((b,), jnp.int32),
            pltpu.VMEM((b,), jnp.int32),
            pltpu.VMEM((b,), jnp.float32),
            pltpu.VMEM((b,), jnp.float32),
            pltpu.VMEM((b,), jnp.float32),
            pltpu.VMEM((_P1_CHUNK,), jnp.float32),
            pltpu.VMEM((1, max(tail, 1)), jnp.float32),
            pltpu.SemaphoreType.DMA,
            pltpu.SemaphoreType.DMA,
            pltpu.SemaphoreType.DMA,
            pltpu.SemaphoreType.DMA,
            pltpu.SemaphoreType.DMA,
        ],
        compiler_params=pltpu.CompilerParams(needs_layout_passes=False),
    )
    def gather(table_hbm, idx_hbm, out_hbm, scr_hbm, idx0_v, idx1_v, vals0_v,
               vals1_v, vals2_v, chunk_v, tail_v, s_idx, s_g0, s_g1, s_w0,
               s_w1):
        idx_b = [idx0_v, idx1_v]
        vals_b = [vals0_v, vals1_v, vals2_v]
        s_g = [s_g0, s_g1]
        s_w = [s_w0, s_w1]
        w = lax.axis_index("s") * _NUM_CORES + lax.axis_index("c")
        row = scr_hbm.at[pl.ds(w * v, v)]
        feat = table_hbm.at[w]

        ci = [None] * l_fields
        for r in range(min(2, l_fields)):
            ci[r] = pltpu.async_copy(idx_hbm.at[r], idx_b[r % 2], s_idx)

        for off, sz in chunks:
            pltpu.sync_copy(
                feat.at[pl.ds(off, sz)], chunk_v.at[pl.ds(0, sz)]
            )
            pltpu.sync_copy(
                chunk_v.at[pl.ds(0, sz)], row.at[pl.ds(off, sz)]
            )
        if tail:
            pltpu.sync_copy(
                table_hbm.at[pl.ds(w, 1), pl.ds(tail_off, tail)], tail_v
            )
            pltpu.sync_copy(tail_v.at[0], row.at[pl.ds(tail_off, tail)])

        cg = [None] * l_fields
        cw = [None] * l_fields
        for r in range(l_fields):
            ci[r].wait()
            if r >= 3:
                cw[r - 3].wait()
            cg[r] = pltpu.async_copy(
                row.at[idx_b[r % 2]], vals_b[r % 3], s_g[r % 2]
            )
            if r >= 1:
                cg[r - 1].wait()
                cw[r - 1] = pltpu.async_copy(
                    vals_b[(r - 1) % 3], out_hbm.at[(r - 1) * d + w],
                    s_w[(r - 1) % 2]
                )
                if r + 1 < l_fields:
                    ci[r + 1] = pltpu.async_copy(
                        idx_hbm.at[r + 1], idx_b[(r + 1) % 2], s_idx
                    )
        lf = l_fields
        cg[lf - 1].wait()
        if lf >= 3:
            cw[lf - 3].wait()
        if lf >= 2:
            cw[lf - 2].wait()
        pltpu.sync_copy(
            vals_b[(lf - 1) % 3], out_hbm.at[(lf - 1) * d + w]
        )

    return gather(table_t, idx_t)[0]


def kernel(inputs, embedding):
    b, l = inputs.shape
    v, d = embedding.shape
    idx_t = inputs.T.astype(jnp.int32)
    table_t = embedding.T
    out_t = _gather_t(table_t, idx_t, v)
    return out_t.T


# speedup vs baseline: 1.2155x; 1.0290x over previous
import functools

import jax
import jax.numpy as jnp
from jax import lax
from jax.experimental import pallas as pl
from jax.experimental.pallas import tpu as pltpu
from jax.experimental.pallas import tpu_sc as plsc

_NUM_CORES = 2
_NUM_SUBCORES = 16
_NUM_WORKERS = _NUM_CORES * _NUM_SUBCORES
_P1_CHUNK = 50944


@functools.partial(jax.jit, static_argnums=(2,))
def _gather_t(table_t, idx_t, v):
    d, _ = table_t.shape
    l_fields, b = idx_t.shape
    m_rows = l_fields * d
    mesh = plsc.VectorSubcoreMesh(core_axis_name="c", subcore_axis_name="s")

    chunks = [(t * _P1_CHUNK, _P1_CHUNK) for t in range(v // _P1_CHUNK)]
    rem_off = (v // _P1_CHUNK) * _P1_CHUNK
    rem_aligned = ((v - rem_off) // 128) * 128
    if rem_aligned:
        chunks.append((rem_off, rem_aligned))
    tail_off = rem_off + rem_aligned
    tail = v - tail_off

    @functools.partial(
        pl.kernel,
        mesh=mesh,
        out_type=[
            jax.ShapeDtypeStruct((m_rows, b), jnp.float32),
            jax.ShapeDtypeStruct((d * v,), jnp.float32),
        ],
        scratch_types=[
            pltpu.VMEM((b,), jnp.int32),
            pltpu.VMEM((b,), jnp.int32),
            pltpu.VMEM((b,), jnp.int32),
            pltpu.VMEM((b,), jnp.float32),
            pltpu.VMEM((b,), jnp.float32),
            pltpu.VMEM((b,), jnp.float32),
            pltpu.VMEM((b,), jnp.float32),
            pltpu.VMEM((_P1_CHUNK,), jnp.float32),
            pltpu.VMEM((_P1_CHUNK,), jnp.float32),
            pltpu.VMEM((1, max(tail, 1)), jnp.float32),
            pltpu.SemaphoreType.DMA,
            pltpu.SemaphoreType.DMA,
            pltpu.SemaphoreType.DMA,
            pltpu.SemaphoreType.DMA,
            pltpu.SemaphoreType.DMA,
            pltpu.SemaphoreType.DMA,
            pltpu.SemaphoreType.DMA,
            pltpu.SemaphoreType.DMA,
            pltpu.SemaphoreType.DMA,
        ],
        compiler_params=pltpu.CompilerParams(needs_layout_passes=False),
    )
    def gather(table_hbm, idx_hbm, out_hbm, scr_hbm, idx0_v, idx1_v, idx2_v,
               vals0_v, vals1_v, vals2_v, vals3_v, chunk0_v, chunk1_v, tail_v,
               s_idx, s_g0, s_g1, s_g2, s_w0, s_w1, s_w2, s_in0, s_in1):
        idx_b = [idx0_v, idx1_v, idx2_v]
        vals_b = [vals0_v, vals1_v, vals2_v, vals3_v]
        chunk_b = [chunk0_v, chunk1_v]
        s_g = [s_g0, s_g1, s_g2]
        s_w = [s_w0, s_w1, s_w2]
        s_in = [s_in0, s_in1]
        w = lax.axis_index("s") * _NUM_CORES + lax.axis_index("c")
        row = scr_hbm.at[pl.ds(w * v, v)]
        feat = table_hbm.at[w]

        ci = [None] * l_fields
        for r in range(min(3, l_fields)):
            ci[r] = pltpu.async_copy(idx_hbm.at[r], idx_b[r % 3], s_idx)

        n_ch = len(chunks)
        cin = [None] * n_ch
        off0, sz0 = chunks[0]
        cin[0] = pltpu.async_copy(
            feat.at[pl.ds(off0, sz0)], chunk_b[0].at[pl.ds(0, sz0)], s_in[0]
        )
        for t in range(n_ch):
            off, sz = chunks[t]
            cin[t].wait()
            if t + 1 < n_ch:
                off2, sz2 = chunks[t + 1]
                cin[t + 1] = pltpu.async_copy(
                    feat.at[pl.ds(off2, sz2)],
                    chunk_b[(t + 1) % 2].at[pl.ds(0, sz2)],
                    s_in[(t + 1) % 2],
                )
            pltpu.sync_copy(
                chunk_b[t % 2].at[pl.ds(0, sz)], row.at[pl.ds(off, sz)]
            )
        if tail:
            pltpu.sync_copy(
                table_hbm.at[pl.ds(w, 1), pl.ds(tail_off, tail)], tail_v
            )
            pltpu.sync_copy(tail_v.at[0], row.at[pl.ds(tail_off, tail)])

        cg = [None] * l_fields
        cw = [None] * l_fields
        for r in range(l_fields):
            ci[r].wait()
            if r >= 4:
                cw[r - 4].wait()
            cg[r] = pltpu.async_copy(
                row.at[idx_b[r % 3]], vals_b[r % 4], s_g[r % 3]
            )
            if r >= 2:
                cg[r - 2].wait()
                cw[r - 2] = pltpu.async_copy(
                    vals_b[(r - 2) % 4], out_hbm.at[(r - 2) * d + w],
                    s_w[(r - 2) % 3]
                )
                if r + 1 < l_fields:
                    ci[r + 1] = pltpu.async_copy(
                        idx_hbm.at[r + 1], idx_b[(r + 1) % 3], s_idx
                    )
        lf = l_fields
        if lf >= 2:
            cg[lf - 2].wait()
            cw[lf - 2] = pltpu.async_copy(
                vals_b[(lf - 2) % 4], out_hbm.at[(lf - 2) * d + w],
                s_w[(lf - 2) % 3]
            )
        cg[lf - 1].wait()
        for q in range(max(lf - 4, 0), lf - 1):
            cw[q].wait()
        pltpu.sync_copy(
            vals_b[(lf - 1) % 4], out_hbm.at[(lf - 1) * d + w]
        )

    return gather(table_t, idx_t)[0]


def kernel(inputs, embedding):
    b, l = inputs.shape
    v, d = embedding.shape
    idx_t = inputs.T.astype(jnp.int32)
    table_t = embedding.T
    out_t = _gather_t(table_t, idx_t, v)
    return out_t.T
